# baseline (speedup 1.0000x reference)
import functools

import jax
import jax.numpy as jnp
import numpy as np
from jax.experimental import pallas as pl
from jax.experimental.pallas import tpu as pltpu

_BLK = 400
_CCH = 2000


def _body(arow_ref, brow_ref, bcol_ref, orow_ref, fin_ref, p2_ref,
          x_ref, w1_ref, b1_ref, w2_ref, b2_ref, adj_ref, adjc_ref,
          o_ref, s1_ref, s2_ref, acc_ref, *, n, ncls, nblk, blk, cch):
    i = pl.program_id(0)

    @pl.when(i == 0)
    def _prologue():
        s1_ref[...] = jnp.dot(x_ref[...], w1_ref[...],
                              preferred_element_type=jnp.float32)

    @pl.when((i >= 1) & (i <= nblk))
    def _pass1():
        r = i - 1
        h = jnp.dot(adj_ref[...], s1_ref[...],
                    preferred_element_type=jnp.float32)
        h = jnp.maximum(h + b1_ref[...], 0.0)
        s2_ref[pl.ds(r * blk, blk), :] = jnp.dot(
            h, w2_ref[...], preferred_element_type=jnp.float32)
        cutoff = (i * blk) // cch * cch
        rows = jax.lax.broadcasted_iota(jnp.int32, (n, ncls), 0)
        s2m = jnp.where(rows < cutoff, s2_ref[...], 0.0)
        acc_ref[pl.ds(r * blk, blk), :] = jnp.dot(
            adj_ref[...], s2m, preferred_element_type=jnp.float32)

    @pl.when(p2_ref[i] == 1)
    def _pass2():
        r = brow_ref[i]
        k = bcol_ref[i]
        part = jnp.dot(adjc_ref[:, 0, 0, :],
                       s2_ref[pl.ds(k * cch, cch), :],
                       preferred_element_type=jnp.float32)
        acc_ref[pl.ds(r * blk, blk), :] = (
            acc_ref[pl.ds(r * blk, blk), :] + part)

    @pl.when(fin_ref[i] == 1)
    def _finalize():
        r = orow_ref[i]
        o = acc_ref[pl.ds(r * blk, blk), :] + b2_ref[...]
        m = jnp.max(o, axis=1, keepdims=True)
        e = o - m
        o_ref[...] = e - jnp.log(jnp.sum(jnp.exp(e), axis=1, keepdims=True))


def kernel(x, adj, W1, b1, W2, b2):
    n, nfeat = x.shape
    nhid = W1.shape[1]
    ncls = W2.shape[1]
    blk, cch = _BLK, _CCH
    nblk = n // blk
    ncch = n // cch

    pairs = [(r, k)
             for r in range(nblk)
             for k in range(((r + 1) * blk) // cch, ncch)]
    covered = {r for r, _ in pairs}
    tail = [r for r in range(nblk) if r not in covered]
    nsteps = 1 + nblk + len(pairs) + len(tail)

    arow = np.zeros(nsteps, np.int32)
    brow = np.zeros(nsteps, np.int32)
    bcol = np.zeros(nsteps, np.int32)
    orow = np.zeros(nsteps, np.int32)
    fin = np.zeros(nsteps, np.int32)
    p2 = np.zeros(nsteps, np.int32)

    for s in range(1, 1 + nblk):
        arow[s] = s - 1
    arow[1 + nblk:] = nblk - 1
    if pairs:
        brow[0:1 + nblk] = pairs[0][0]
        bcol[0:1 + nblk] = pairs[0][1]
        brow[1 + nblk + len(pairs):] = pairs[-1][0]
        bcol[1 + nblk + len(pairs):] = pairs[-1][1]
    for s, (r, k) in enumerate(pairs, start=1 + nblk):
        brow[s], bcol[s], orow[s], p2[s] = r, k, r, 1
        if k == ncch - 1:
            fin[s] = 1
    for s, r in enumerate(tail, start=1 + nblk + len(pairs)):
        orow[s], fin[s] = r, 1

    body = functools.partial(_body, n=n, ncls=ncls, nblk=nblk,
                             blk=blk, cch=cch)

    grid_spec = pltpu.PrefetchScalarGridSpec(
        num_scalar_prefetch=6,
        grid=(nsteps,),
        in_specs=[
            pl.BlockSpec((n, nfeat), lambda i, *s: (0, 0)),
            pl.BlockSpec((nfeat, nhid), lambda i, *s: (0, 0)),
            pl.BlockSpec((1, nhid), lambda i, *s: (0, 0)),
            pl.BlockSpec((nhid, ncls), lambda i, *s: (0, 0)),
            pl.BlockSpec((1, ncls), lambda i, *s: (0, 0)),
            pl.BlockSpec((blk, n),
                         lambda i, ar, *s: (ar[i], 0)),
            pl.BlockSpec((blk, 1, 1, cch),
                         lambda i, ar, br, bc, *s: (br[i], bc[i], 0, 0)),
        ],
        out_specs=pl.BlockSpec((blk, ncls),
                               lambda i, ar, br, bc, orw, *s: (orw[i], 0)),
        scratch_shapes=[
            pltpu.VMEM((n, nhid), jnp.float32),
            pltpu.VMEM((n, ncls), jnp.float32),
            pltpu.VMEM((n, ncls), jnp.float32),
        ],
    )

    return pl.pallas_call(
        body,
        grid_spec=grid_spec,
        out_shape=jax.ShapeDtypeStruct((n, ncls), jnp.float32),
        compiler_params=pltpu.CompilerParams(
            dimension_semantics=("arbitrary",),
        ),
    )(jnp.asarray(arow), jnp.asarray(brow), jnp.asarray(bcol),
      jnp.asarray(orow), jnp.asarray(fin), jnp.asarray(p2),
      x, W1, b1.reshape(1, nhid), W2, b2.reshape(1, ncls), adj,
      adj.reshape(n, ncch, 1, cch))


# speedup vs baseline: 9.3457x; 9.3457x over previous
import functools

import jax
import jax.numpy as jnp
import numpy as np
from jax.experimental import pallas as pl
from jax.experimental.pallas import tpu as pltpu

_BLK = 400
_CCH = 2048


def _body(arow_ref, brow_ref, bcol_ref, orow_ref, fin_ref, p2_ref, edge_ref,
          x_ref, w1_ref, b1_ref, w2_ref, b2_ref, adj_ref, adjc_ref,
          o_ref, s1_ref, s2_ref, s2a_ref, acc_ref,
          *, n, ncls, nblk, blk, cch, ncch, npad):
    i = pl.program_id(0)

    @pl.when(i == 0)
    def _prologue():
        s1_ref[...] = jnp.dot(x_ref[...], w1_ref[...],
                              preferred_element_type=jnp.float32)
        s2a_ref[...] = jnp.zeros((npad, ncls), jnp.float32)
        s2_ref[pl.ds(n, npad - n), :] = jnp.zeros((npad - n, ncls),
                                                  jnp.float32)

    @pl.when((i >= 1) & (i <= nblk))
    def _pass1():
        r = i - 1
        h = jnp.dot(adj_ref[...], s1_ref[...],
                    preferred_element_type=jnp.float32)
        h = jnp.maximum(h + b1_ref[...], 0.0)
        s2_ref[pl.ds(r * blk, blk), :] = jnp.dot(
            h, w2_ref[...], preferred_element_type=jnp.float32)
        cutoff = (i * blk) // cch * cch
        prev = ((i - 1) * blk) // cch * cch

        @pl.when(cutoff > prev)
        def _advance():
            s2a_ref[pl.ds(prev, cch), :] = s2_ref[pl.ds(prev, cch), :]

        acc_ref[pl.ds(r * blk, blk), :] = jnp.dot(
            adj_ref[...], s2a_ref[pl.ds(0, n), :],
            preferred_element_type=jnp.float32)

    @pl.when(p2_ref[i] == 1)
    def _pass2():
        r = brow_ref[i]
        k = bcol_ref[i]
        win = s2_ref[pl.ds(k * cch, cch), :]

        @pl.when(edge_ref[i] == 0)
        def _full():
            part = jnp.dot(adjc_ref[...], win,
                           preferred_element_type=jnp.float32)
            acc_ref[pl.ds(r * blk, blk), :] = (
                acc_ref[pl.ds(r * blk, blk), :] + part)

        @pl.when(edge_ref[i] == 1)
        def _edge():
            cols = jax.lax.broadcasted_iota(jnp.int32, (blk, cch), 1)
            a = jnp.where(cols < n - (ncch - 1) * cch, adjc_ref[...], 0.0)
            part = jnp.dot(a, win, preferred_element_type=jnp.float32)
            acc_ref[pl.ds(r * blk, blk), :] = (
                acc_ref[pl.ds(r * blk, blk), :] + part)

    @pl.when(fin_ref[i] == 1)
    def _finalize():
        r = orow_ref[i]
        o = acc_ref[pl.ds(r * blk, blk), :] + b2_ref[...]
        m = jnp.max(o, axis=1, keepdims=True)
        e = o - m
        o_ref[...] = e - jnp.log(jnp.sum(jnp.exp(e), axis=1, keepdims=True))


def kernel(x, adj, W1, b1, W2, b2):
    n, nfeat = x.shape
    nhid = W1.shape[1]
    ncls = W2.shape[1]
    blk, cch = _BLK, _CCH
    nblk = n // blk
    ncch = -(-n // cch)
    npad = ncch * cch

    pairs = [(r, k)
             for r in range(nblk)
             for k in range(((r + 1) * blk) // cch, ncch)]
    covered = {r for r, _ in pairs}
    tail = [r for r in range(nblk) if r not in covered]
    nsteps = 1 + nblk + len(pairs) + len(tail)

    arow = np.zeros(nsteps, np.int32)
    brow = np.zeros(nsteps, np.int32)
    bcol = np.zeros(nsteps, np.int32)
    orow = np.zeros(nsteps, np.int32)
    fin = np.zeros(nsteps, np.int32)
    p2 = np.zeros(nsteps, np.int32)
    edge = np.zeros(nsteps, np.int32)

    for s in range(1, 1 + nblk):
        arow[s] = s - 1
    arow[1 + nblk:] = nblk - 1
    if pairs:
        brow[0:1 + nblk] = pairs[0][0]
        bcol[0:1 + nblk] = pairs[0][1]
        brow[1 + nblk + len(pairs):] = pairs[-1][0]
        bcol[1 + nblk + len(pairs):] = pairs[-1][1]
    for s, (r, k) in enumerate(pairs, start=1 + nblk):
        brow[s], bcol[s], orow[s], p2[s] = r, k, r, 1
        if k == ncch - 1:
            fin[s] = 1
            if ncch * cch > n:
                edge[s] = 1
    for s, r in enumerate(tail, start=1 + nblk + len(pairs)):
        orow[s], fin[s] = r, 1

    body = functools.partial(_body, n=n, ncls=ncls, nblk=nblk,
                             blk=blk, cch=cch, ncch=ncch, npad=npad)

    grid_spec = pltpu.PrefetchScalarGridSpec(
        num_scalar_prefetch=7,
        grid=(nsteps,),
        in_specs=[
            pl.BlockSpec((n, nfeat), lambda i, *s: (0, 0)),
            pl.BlockSpec((nfeat, nhid), lambda i, *s: (0, 0)),
            pl.BlockSpec((1, nhid), lambda i, *s: (0, 0)),
            pl.BlockSpec((nhid, ncls), lambda i, *s: (0, 0)),
            pl.BlockSpec((1, ncls), lambda i, *s: (0, 0)),
            pl.BlockSpec((blk, n),
                         lambda i, ar, *s: (ar[i], 0)),
            pl.BlockSpec((blk, cch),
                         lambda i, ar, br, bc, *s: (br[i], bc[i])),
        ],
        out_specs=pl.BlockSpec((blk, ncls),
                               lambda i, ar, br, bc, orw, *s: (orw[i], 0)),
        scratch_shapes=[
            pltpu.VMEM((n, nhid), jnp.float32),
            pltpu.VMEM((npad, ncls), jnp.float32),
            pltpu.VMEM((npad, ncls), jnp.float32),
            pltpu.VMEM((n, ncls), jnp.float32),
        ],
    )

    return pl.pallas_call(
        body,
        grid_spec=grid_spec,
        out_shape=jax.ShapeDtypeStruct((n, ncls), jnp.float32),
        compiler_params=pltpu.CompilerParams(
            dimension_semantics=("arbitrary",),
            vmem_limit_bytes=67108864,
        ),
    )(jnp.asarray(arow), jnp.asarray(brow), jnp.asarray(bcol),
      jnp.asarray(orow), jnp.asarray(fin), jnp.asarray(p2),
      jnp.asarray(edge),
      x, W1, b1.reshape(1, nhid), W2, b2.reshape(1, ncls), adj, adj)


# speedup vs baseline: 12.5847x; 1.3466x over previous
import functools

import jax
import jax.numpy as jnp
from jax.experimental import pallas as pl
from jax.experimental.pallas import tpu as pltpu

_BLK = 400
_NSTREAM = 5


def _body(x_ref, w1_ref, b1_ref, w2_ref, b2_ref, a0_ref, a1_ref, a2_ref,
          a3_ref, a4_ref, o_ref, s1_ref, s2_ref, *, nblk, blk, sub):
    i = pl.program_id(0)
    adj_refs = (a0_ref, a1_ref, a2_ref, a3_ref, a4_ref)

    @pl.when(i == 0)
    def _prologue():
        s1_ref[...] = jnp.dot(x_ref[...], w1_ref[...],
                              preferred_element_type=jnp.float32)

    @pl.when((i >= 1) & (i <= nblk))
    def _pass1():
        r = i - 1
        for q, aq in enumerate(adj_refs):
            h = jnp.dot(aq[...], s1_ref[...],
                        preferred_element_type=jnp.float32)
            h = jnp.maximum(h + b1_ref[...], 0.0)
            s2_ref[pl.ds(r * blk + q * sub, sub), :] = jnp.dot(
                h, w2_ref[...], preferred_element_type=jnp.float32)

    @pl.when(i > nblk)
    def _pass2():
        for q, aq in enumerate(adj_refs):
            o = jnp.dot(aq[...], s2_ref[...],
                        preferred_element_type=jnp.float32)
            o = o + b2_ref[...]
            m = jnp.max(o, axis=1, keepdims=True)
            e = o - m
            o_ref[pl.ds(q * sub, sub), :] = e - jnp.log(
                jnp.sum(jnp.exp(e), axis=1, keepdims=True))


def kernel(x, adj, W1, b1, W2, b2):
    n, nfeat = x.shape
    nhid = W1.shape[1]
    ncls = W2.shape[1]
    blk = _BLK
    sub = blk // _NSTREAM
    nblk = n // blk

    body = functools.partial(_body, nblk=nblk, blk=blk, sub=sub)

    def stream_idx(q):
        def idx(i):
            r = jnp.where(i <= nblk, jnp.maximum(i - 1, 0), i - nblk - 1)
            return (r * _NSTREAM + q, 0)
        return idx

    def out_idx(i):
        return (jnp.maximum(i - nblk - 1, 0), 0)

    return pl.pallas_call(
        body,
        grid=(1 + 2 * nblk,),
        in_specs=[
            pl.BlockSpec((n, nfeat), lambda i: (0, 0)),
            pl.BlockSpec((nfeat, nhid), lambda i: (0, 0)),
            pl.BlockSpec((1, nhid), lambda i: (0, 0)),
            pl.BlockSpec((nhid, ncls), lambda i: (0, 0)),
            pl.BlockSpec((1, ncls), lambda i: (0, 0)),
            pl.BlockSpec((sub, n), stream_idx(0)),
            pl.BlockSpec((sub, n), stream_idx(1)),
            pl.BlockSpec((sub, n), stream_idx(2)),
            pl.BlockSpec((sub, n), stream_idx(3)),
            pl.BlockSpec((sub, n), stream_idx(4)),
        ],
        out_specs=pl.BlockSpec((blk, ncls), out_idx),
        out_shape=jax.ShapeDtypeStruct((n, ncls), jnp.float32),
        scratch_shapes=[
            pltpu.VMEM((n, nhid), jnp.float32),
            pltpu.VMEM((n, ncls), jnp.float32),
        ],
        compiler_params=pltpu.CompilerParams(
            dimension_semantics=("arbitrary",),
            vmem_limit_bytes=67108864,
        ),
    )(x, W1, b1.reshape(1, nhid), W2, b2.reshape(1, ncls),
      adj, adj, adj, adj, adj)
